# baseline (speedup 1.0000x reference)
import functools

import jax
import jax.numpy as jnp
from jax import lax
from jax.experimental import pallas as pl
from jax.experimental.pallas import tpu as pltpu
from jax.experimental.pallas import tpu_sc as plsc

L = 16
NC = 2
NS = 16
NW = NC * NS
B = 16384
D = 64
BPW = B // NW
GCH = 128
NCH = BPW // GCH
TB = 1024


def _gather_prod_body(uidx_hbm, iidx_hbm, utab_hbm, itab_hbm, out_hbm,
                      uidx_v, iidx_v, urows_v, irows_v, gsem):
    wid = lax.axis_index("s") * NC + lax.axis_index("c")
    base = wid * BPW

    pltpu.sync_copy(uidx_hbm.at[pl.ds(base, BPW)], uidx_v)
    pltpu.sync_copy(iidx_hbm.at[pl.ds(base, BPW)], iidx_v)

    copies = []
    for c in range(NCH):
        sl = pl.ds(c * GCH, GCH)
        copies.append(pltpu.async_copy(
            utab_hbm.at[uidx_v.at[sl]], urows_v.at[sl], gsem))
        copies.append(pltpu.async_copy(
            itab_hbm.at[iidx_v.at[sl]], irows_v.at[sl], gsem))
    for cp in copies:
        cp.wait()

    def prod_body(i, carry):
        for c in range(D // L):
            sl = pl.ds(c * L, L)
            urows_v[i, sl] = urows_v[i, sl] * irows_v[i, sl]
        return carry

    lax.fori_loop(0, BPW, prod_body, 0)

    pltpu.sync_copy(urows_v, out_hbm.at[pl.ds(base, BPW)])


def _head_body(x_ref, w_ref, b_ref, o_ref):
    y = jnp.dot(x_ref[...], w_ref[...],
                preferred_element_type=jnp.float32) + b_ref[0, 0]
    o_ref[...] = 1.0 / (1.0 + jnp.exp(-y))


@functools.partial(jax.jit, static_argnames=())
def _gmf(uidx, iidx, utab, itab, w2d, b2d):
    mesh = plsc.VectorSubcoreMesh(core_axis_name="c", subcore_axis_name="s")
    prod = pl.kernel(
        _gather_prod_body,
        mesh=mesh,
        compiler_params=pltpu.CompilerParams(use_tc_tiling_on_sc=False),
        out_type=jax.ShapeDtypeStruct((B, D), jnp.float32),
        scratch_types=[
            pltpu.VMEM((BPW,), jnp.int32),
            pltpu.VMEM((BPW,), jnp.int32),
            pltpu.VMEM((BPW, D), jnp.float32),
            pltpu.VMEM((BPW, D), jnp.float32),
            pltpu.SemaphoreType.DMA,
        ],
    )(uidx, iidx, utab, itab)
    return pl.pallas_call(
        _head_body,
        grid=(B // TB,),
        in_specs=[
            pl.BlockSpec((TB, D), lambda i: (i, 0)),
            pl.BlockSpec((D, 1), lambda i: (0, 0)),
            pl.BlockSpec((1, 1), lambda i: (0, 0)),
        ],
        out_specs=pl.BlockSpec((TB, 1), lambda i: (i, 0)),
        out_shape=jax.ShapeDtypeStruct((B, 1), jnp.float32),
    )(prod, w2d, b2d)


def kernel(user_indices, item_indices, user_table, item_table, W, b):
    w2d = jnp.reshape(W.astype(jnp.float32), (D, 1))
    b2d = jnp.reshape(b.astype(jnp.float32), (1, 1))
    return _gmf(user_indices.astype(jnp.int32), item_indices.astype(jnp.int32),
                user_table, item_table, w2d, b2d)
